# baseline (speedup 1.0000x reference)
import functools

import jax
import jax.numpy as jnp
from jax import lax
from jax.experimental import pallas as pl
from jax.experimental.pallas import tpu as pltpu
from jax.experimental.pallas import tpu_sc as plsc

N = 10000
E = 320000
D = 128
NC, NS = 2, 16
NW = NC * NS
CH = 128
NCHUNK = E // CH
NBUF = 3
KMAX = 81

ROWS_A, ROWS_B = 632, 520

_mesh = plsc.VectorSubcoreMesh(core_axis_name="c", subcore_axis_name="s")


def _tile_rows(s):
    return (s * ROWS_A, ROWS_A if s < NS - 1 else ROWS_B)


@functools.partial(
    pl.kernel,
    out_type=(jax.ShapeDtypeStruct((NC * N,), jnp.float32),
              jax.ShapeDtypeStruct((NC * N,), jnp.float32)),
    mesh=_mesh,
    scratch_types=[
        pltpu.VMEM((NBUF, CH), jnp.int32),
        pltpu.VMEM((NBUF, CH), jnp.int32),
        pltpu.VMEM((CH,), jnp.float32),
        pltpu.VMEM((ROWS_A + 8,), jnp.float32),
        pltpu.VMEM_SHARED((N,), jnp.float32),
        pltpu.VMEM_SHARED((N,), jnp.float32),
        pltpu.SemaphoreType.DMA((NBUF,)),
        pltpu.SemaphoreType.DMA((NBUF,)),
        pltpu.SemaphoreType.DMA((NBUF,)),
        pltpu.SemaphoreType.DMA((NBUF,)),
    ],
)
def _hist(src_h, dst_h, dout_h, din_h, src_v, dst_v, ones_v, stage_v,
          dout_sh, din_sh, sem_i, sem_j, sem_h, sem_n):
    c = lax.axis_index("c")
    s = lax.axis_index("s")
    w = s * NC + c

    one16 = jnp.ones((16,), jnp.float32)
    zero16 = jnp.zeros((16,), jnp.float32)
    for i in range(CH // 16):
        ones_v[pl.ds(i * 16, 16)] = one16
    for i in range((ROWS_A + 8) // 16):
        stage_v[pl.ds(i * 16, 16)] = zero16

    def valid(k):
        return (k >= 0) & (k < KMAX) & (k * NW + w < NCHUNK)

    def issue_idx(k, p):
        @pl.when(valid(k))
        def _():
            pltpu.async_copy(src_h.at[k * NW + w], src_v.at[p], sem_i.at[p])
            pltpu.async_copy(dst_h.at[k * NW + w], dst_v.at[p], sem_j.at[p])

    issue_idx(0, 0)

    for sv in range(NS):
        lo, sz = _tile_rows(sv)

        @pl.when(s == sv)
        def _():
            pltpu.sync_copy(stage_v.at[pl.ds(0, sz)], dout_sh.at[pl.ds(lo, sz)])
            pltpu.sync_copy(stage_v.at[pl.ds(0, sz)], din_sh.at[pl.ds(lo, sz)])

    plsc.subcore_barrier()

    def wait_idx(k, p):
        @pl.when(valid(k))
        def _():
            pltpu.make_async_copy(src_h.at[k * NW + w], src_v.at[p],
                                  sem_i.at[p]).wait()
            pltpu.make_async_copy(dst_h.at[k * NW + w], dst_v.at[p],
                                  sem_j.at[p]).wait()

    def issue_hist(k, p):
        @pl.when(valid(k))
        def _():
            pltpu.async_copy(ones_v, dout_sh.at[src_v.at[p]], sem_h.at[p],
                             add=True)
            pltpu.async_copy(ones_v, din_sh.at[dst_v.at[p]], sem_n.at[p],
                             add=True)

    def wait_hist(k, p):
        @pl.when(valid(k))
        def _():
            pltpu.make_async_copy(ones_v, dout_sh.at[src_v.at[p]],
                                  sem_h.at[p]).wait()
            pltpu.make_async_copy(ones_v, din_sh.at[dst_v.at[p]],
                                  sem_n.at[p]).wait()

    def body(i, carry):
        for sub in range(NBUF):
            k = i * NBUF + sub
            p = sub
            wait_idx(k, p)
            issue_hist(k, p)
            wait_hist(k - 2, (p + 1) % NBUF)
            issue_idx(k + 1, (p + 1) % NBUF)
        return carry

    lax.fori_loop(0, KMAX // NBUF, body, 0)
    for kk in range(KMAX - 2, KMAX):
        wait_hist(kk, kk % NBUF)

    plsc.subcore_barrier()

    for sv in range(NS):
        lo, sz = _tile_rows(sv)

        @pl.when(s == sv)
        def _():
            pltpu.sync_copy(dout_sh.at[pl.ds(lo, sz)], stage_v.at[pl.ds(0, sz)])
            pltpu.sync_copy(stage_v.at[pl.ds(0, sz)],
                            dout_h.at[pl.ds(c * N + lo, sz)])
            pltpu.sync_copy(din_sh.at[pl.ds(lo, sz)], stage_v.at[pl.ds(0, sz)])
            pltpu.sync_copy(stage_v.at[pl.ds(0, sz)],
                            din_h.at[pl.ds(c * N + lo, sz)])


@functools.partial(
    pl.kernel,
    out_type=jax.ShapeDtypeStruct((NC, N, D), jnp.float32),
    mesh=_mesh,
    scratch_types=[
        pltpu.VMEM((NBUF, CH), jnp.int32),
        pltpu.VMEM((NBUF, CH), jnp.int32),
        pltpu.VMEM((NBUF, CH, D), jnp.float32),
        pltpu.VMEM_SHARED((N, D), jnp.float32),
        pltpu.SemaphoreType.DMA((NBUF,)),
        pltpu.SemaphoreType.DMA((NBUF,)),
        pltpu.SemaphoreType.DMA((NBUF,)),
        pltpu.SemaphoreType.DMA((NBUF,)),
    ],
)
def _agg(zp_h, src_h, dst_h, out_h, src_v, dst_v, rows_v, agg_sh,
         sem_s, sem_d, sem_g, sem_a):
    c = lax.axis_index("c")
    s = lax.axis_index("s")
    w = s * NC + c

    zero16 = jnp.zeros((16,), jnp.float32)
    for r in range(8):
        for q in range(D // 16):
            rows_v[0, r, pl.ds(q * 16, 16)] = zero16

    def valid(k):
        return (k >= 0) & (k < KMAX) & (k * NW + w < NCHUNK)

    def issue_idx(k, p):
        @pl.when(valid(k))
        def _():
            pltpu.async_copy(src_h.at[k * NW + w], src_v.at[p], sem_s.at[p])
            pltpu.async_copy(dst_h.at[k * NW + w], dst_v.at[p], sem_d.at[p])

    issue_idx(0, 0)

    for sv in range(NS):
        lo, sz = _tile_rows(sv)

        @pl.when(s == sv)
        def _():
            def zbody(i, carry):
                pltpu.sync_copy(rows_v.at[0, pl.ds(0, 8)],
                                agg_sh.at[pl.ds(lo + i * 8, 8)])
                return carry

            lax.fori_loop(0, sz // 8, zbody, 0)

    plsc.subcore_barrier()

    def wait_idx(k, p):
        @pl.when(valid(k))
        def _():
            pltpu.make_async_copy(src_h.at[k * NW + w], src_v.at[p],
                                  sem_s.at[p]).wait()
            pltpu.make_async_copy(dst_h.at[k * NW + w], dst_v.at[p],
                                  sem_d.at[p]).wait()

    def issue_gather(k, p):
        @pl.when(valid(k))
        def _():
            pltpu.async_copy(zp_h.at[src_v.at[p]], rows_v.at[p], sem_g.at[p])

    def wait_gather(k, p):
        @pl.when(valid(k))
        def _():
            pltpu.make_async_copy(zp_h.at[src_v.at[p]], rows_v.at[p],
                                  sem_g.at[p]).wait()

    def issue_scat(k, p):
        @pl.when(valid(k))
        def _():
            pltpu.async_copy(rows_v.at[p], agg_sh.at[dst_v.at[p]], sem_a.at[p],
                             add=True)

    def wait_scat(k, p):
        @pl.when(valid(k))
        def _():
            pltpu.make_async_copy(rows_v.at[p], agg_sh.at[dst_v.at[p]],
                                  sem_a.at[p]).wait()

    def body(i, carry):
        for sub in range(NBUF):
            k = i * NBUF + sub
            p = sub
            pn = (p + 1) % NBUF
            pb = (p + 2) % NBUF
            wait_idx(k, p)
            issue_gather(k, p)
            wait_gather(k - 1, pb)
            issue_scat(k - 1, pb)
            wait_scat(k - 2, pn)
            issue_idx(k + 1, pn)
        return carry

    lax.fori_loop(0, KMAX // NBUF, body, 0)
    kk = KMAX - 1
    wait_gather(kk, kk % NBUF)
    issue_scat(kk, kk % NBUF)
    for kk in range(KMAX - 2, KMAX):
        wait_scat(kk, kk % NBUF)

    plsc.subcore_barrier()

    for sv in range(NS):
        lo, sz = _tile_rows(sv)

        @pl.when(s == sv)
        def _():
            pltpu.sync_copy(agg_sh.at[pl.ds(lo, sz)],
                            out_h.at[c, pl.ds(lo, sz)])


def _dense_pre_body(x_ref, w_ref, dparts_ref, o_ref):
    deg = dparts_ref[0, :] + dparts_ref[1, :]
    ns = lax.rsqrt(jnp.maximum(deg, 1.0))
    z = jnp.dot(x_ref[...], w_ref[...], preferred_element_type=jnp.float32)
    o_ref[...] = z * ns[:, None]


_dense_pre = pl.pallas_call(
    _dense_pre_body,
    out_shape=jax.ShapeDtypeStruct((N, D), jnp.float32),
)


def _dense_post_body(aggp_ref, dinp_ref, b_ref, a1_ref, g_ref, be_ref, a2_ref,
                     o_ref):
    agg = aggp_ref[0] + aggp_ref[1]
    deg = dinp_ref[0, :] + dinp_ref[1, :]
    nd = lax.rsqrt(jnp.maximum(deg, 1.0))
    out = agg * nd[:, None] + b_ref[...]
    a1 = a1_ref[0]
    out = jnp.where(out >= 0, out, a1 * out)
    mean = jnp.mean(out, axis=0)
    cent = out - mean
    var = jnp.mean(cent * cent, axis=0)
    hn = cent * lax.rsqrt(var + 1e-5) * g_ref[...] + be_ref[...]
    a2 = a2_ref[0]
    o_ref[...] = jnp.where(hn >= 0, hn, a2 * hn)


_dense_post = pl.pallas_call(
    _dense_post_body,
    out_shape=jax.ShapeDtypeStruct((N, D), jnp.float32),
)


def kernel(heat, edge_index, W, b, a1, gamma, beta, a2):
    ei = edge_index.astype(jnp.int32)
    src_r = ei[0].reshape(NCHUNK, CH)
    dst_r = ei[1].reshape(NCHUNK, CH)
    dout_p, din_p = _hist(src_r, dst_r)
    z = _dense_pre(heat, W, dout_p.reshape(NC, N))
    agg_p = _agg(z, src_r, dst_r)
    return _dense_post(agg_p, din_p.reshape(NC, N), b, a1, gamma, beta, a2)


# speedup vs baseline: 1.1140x; 1.1140x over previous
import functools

import jax
import jax.numpy as jnp
from jax import lax
from jax.experimental import pallas as pl
from jax.experimental.pallas import tpu as pltpu
from jax.experimental.pallas import tpu_sc as plsc

N = 10000
E = 320000
D = 128
NC, NS = 2, 16
NW = NC * NS
CH = 128
NCHUNK = E // CH
NBUF = 3
KMAX = 81

ROWS_A, ROWS_B = 632, 520

_mesh = plsc.VectorSubcoreMesh(core_axis_name="c", subcore_axis_name="s")


def _tile_rows(s):
    return (s * ROWS_A, ROWS_A if s < NS - 1 else ROWS_B)


@functools.partial(
    pl.kernel,
    out_type=(jax.ShapeDtypeStruct((NC * N,), jnp.float32),
              jax.ShapeDtypeStruct((NC * N,), jnp.float32)),
    mesh=_mesh,
    scratch_types=[
        pltpu.VMEM((NBUF, CH), jnp.int32),
        pltpu.VMEM((NBUF, CH), jnp.int32),
        pltpu.VMEM((CH,), jnp.float32),
        pltpu.VMEM((ROWS_A + 8,), jnp.float32),
        pltpu.VMEM_SHARED((N,), jnp.float32),
        pltpu.VMEM_SHARED((N,), jnp.float32),
        pltpu.SemaphoreType.DMA((NBUF,)),
        pltpu.SemaphoreType.DMA((NBUF,)),
        pltpu.SemaphoreType.DMA((NBUF,)),
        pltpu.SemaphoreType.DMA((NBUF,)),
    ],
)
def _hist(src_h, dst_h, dout_h, din_h, src_v, dst_v, ones_v, stage_v,
          dout_sh, din_sh, sem_i, sem_j, sem_h, sem_n):
    c = lax.axis_index("c")
    s = lax.axis_index("s")
    w = s * NC + c

    one16 = jnp.ones((16,), jnp.float32)
    zero16 = jnp.zeros((16,), jnp.float32)
    for i in range(CH // 16):
        ones_v[pl.ds(i * 16, 16)] = one16
    for i in range((ROWS_A + 8) // 16):
        stage_v[pl.ds(i * 16, 16)] = zero16

    def valid(k):
        return (k >= 0) & (k < KMAX) & (k * NW + w < NCHUNK)

    def issue_idx(k, p):
        @pl.when(valid(k))
        def _():
            pltpu.async_copy(src_h.at[k * NW + w], src_v.at[p], sem_i.at[p])
            pltpu.async_copy(dst_h.at[k * NW + w], dst_v.at[p], sem_j.at[p])

    issue_idx(0, 0)

    for sv in range(NS):
        lo, sz = _tile_rows(sv)

        @pl.when(s == sv)
        def _():
            pltpu.sync_copy(stage_v.at[pl.ds(0, sz)], dout_sh.at[pl.ds(lo, sz)])
            pltpu.sync_copy(stage_v.at[pl.ds(0, sz)], din_sh.at[pl.ds(lo, sz)])

    plsc.subcore_barrier()

    def wait_idx(k, p):
        @pl.when(valid(k))
        def _():
            pltpu.make_async_copy(src_h.at[k * NW + w], src_v.at[p],
                                  sem_i.at[p]).wait()
            pltpu.make_async_copy(dst_h.at[k * NW + w], dst_v.at[p],
                                  sem_j.at[p]).wait()

    def issue_hist(k, p):
        @pl.when(valid(k))
        def _():
            pltpu.async_copy(ones_v, dout_sh.at[src_v.at[p]], sem_h.at[p],
                             add=True)
            pltpu.async_copy(ones_v, din_sh.at[dst_v.at[p]], sem_n.at[p],
                             add=True)

    def wait_hist(k, p):
        @pl.when(valid(k))
        def _():
            pltpu.make_async_copy(ones_v, dout_sh.at[src_v.at[p]],
                                  sem_h.at[p]).wait()
            pltpu.make_async_copy(ones_v, din_sh.at[dst_v.at[p]],
                                  sem_n.at[p]).wait()

    def body(i, carry):
        for sub in range(NBUF):
            k = i * NBUF + sub
            p = sub
            wait_hist(k - 2, (p + 1) % NBUF)
            issue_idx(k + 1, (p + 1) % NBUF)
            wait_idx(k, p)
            issue_hist(k, p)
        return carry

    lax.fori_loop(0, KMAX // NBUF, body, 0)
    for kk in range(KMAX - 2, KMAX):
        wait_hist(kk, kk % NBUF)

    plsc.subcore_barrier()

    for sv in range(NS):
        lo, sz = _tile_rows(sv)

        @pl.when(s == sv)
        def _():
            pltpu.sync_copy(dout_sh.at[pl.ds(lo, sz)], stage_v.at[pl.ds(0, sz)])
            pltpu.sync_copy(stage_v.at[pl.ds(0, sz)],
                            dout_h.at[pl.ds(c * N + lo, sz)])
            pltpu.sync_copy(din_sh.at[pl.ds(lo, sz)], stage_v.at[pl.ds(0, sz)])
            pltpu.sync_copy(stage_v.at[pl.ds(0, sz)],
                            din_h.at[pl.ds(c * N + lo, sz)])


@functools.partial(
    pl.kernel,
    out_type=jax.ShapeDtypeStruct((NC, N, D), jnp.float32),
    mesh=_mesh,
    scratch_types=[
        pltpu.VMEM((NBUF, CH), jnp.int32),
        pltpu.VMEM((NBUF, CH), jnp.int32),
        pltpu.VMEM((NBUF, CH, D), jnp.float32),
        pltpu.VMEM_SHARED((N, D), jnp.float32),
        pltpu.SemaphoreType.DMA((NBUF,)),
        pltpu.SemaphoreType.DMA((NBUF,)),
        pltpu.SemaphoreType.DMA((NBUF,)),
        pltpu.SemaphoreType.DMA((NBUF,)),
    ],
)
def _agg(zp_h, src_h, dst_h, out_h, src_v, dst_v, rows_v, agg_sh,
         sem_s, sem_d, sem_g, sem_a):
    c = lax.axis_index("c")
    s = lax.axis_index("s")
    w = s * NC + c

    zero16 = jnp.zeros((16,), jnp.float32)
    for r in range(8):
        for q in range(D // 16):
            rows_v[0, r, pl.ds(q * 16, 16)] = zero16

    def valid(k):
        return (k >= 0) & (k < KMAX) & (k * NW + w < NCHUNK)

    def issue_idx(k, p):
        @pl.when(valid(k))
        def _():
            pltpu.async_copy(src_h.at[k * NW + w], src_v.at[p], sem_s.at[p])
            pltpu.async_copy(dst_h.at[k * NW + w], dst_v.at[p], sem_d.at[p])

    issue_idx(0, 0)

    for sv in range(NS):
        lo, sz = _tile_rows(sv)

        @pl.when(s == sv)
        def _():
            def zbody(i, carry):
                pltpu.sync_copy(rows_v.at[0, pl.ds(0, 8)],
                                agg_sh.at[pl.ds(lo + i * 8, 8)])
                return carry

            lax.fori_loop(0, sz // 8, zbody, 0)

    plsc.subcore_barrier()

    def wait_idx(k, p):
        @pl.when(valid(k))
        def _():
            pltpu.make_async_copy(src_h.at[k * NW + w], src_v.at[p],
                                  sem_s.at[p]).wait()
            pltpu.make_async_copy(dst_h.at[k * NW + w], dst_v.at[p],
                                  sem_d.at[p]).wait()

    def issue_gather(k, p):
        @pl.when(valid(k))
        def _():
            pltpu.async_copy(zp_h.at[src_v.at[p]], rows_v.at[p], sem_g.at[p])

    def wait_gather(k, p):
        @pl.when(valid(k))
        def _():
            pltpu.make_async_copy(zp_h.at[src_v.at[p]], rows_v.at[p],
                                  sem_g.at[p]).wait()

    def issue_scat(k, p):
        @pl.when(valid(k))
        def _():
            pltpu.async_copy(rows_v.at[p], agg_sh.at[dst_v.at[p]], sem_a.at[p],
                             add=True)

    def wait_scat(k, p):
        @pl.when(valid(k))
        def _():
            pltpu.make_async_copy(rows_v.at[p], agg_sh.at[dst_v.at[p]],
                                  sem_a.at[p]).wait()

    def body(i, carry):
        for sub in range(NBUF):
            k = i * NBUF + sub
            p = sub
            pn = (p + 1) % NBUF
            pb = (p + 2) % NBUF
            wait_scat(k - 2, pn)
            issue_idx(k + 1, pn)
            wait_idx(k, p)
            issue_gather(k, p)
            wait_gather(k - 1, pb)
            issue_scat(k - 1, pb)
        return carry

    lax.fori_loop(0, KMAX // NBUF, body, 0)
    kk = KMAX - 1
    wait_gather(kk, kk % NBUF)
    issue_scat(kk, kk % NBUF)
    for kk in range(KMAX - 2, KMAX):
        wait_scat(kk, kk % NBUF)

    plsc.subcore_barrier()

    for sv in range(NS):
        lo, sz = _tile_rows(sv)

        @pl.when(s == sv)
        def _():
            pltpu.sync_copy(agg_sh.at[pl.ds(lo, sz)],
                            out_h.at[c, pl.ds(lo, sz)])


def _dense_pre_body(x_ref, w_ref, dparts_ref, o_ref):
    deg = dparts_ref[0, :] + dparts_ref[1, :]
    ns = lax.rsqrt(jnp.maximum(deg, 1.0))
    z = jnp.dot(x_ref[...], w_ref[...], preferred_element_type=jnp.float32)
    o_ref[...] = z * ns[:, None]


_dense_pre = pl.pallas_call(
    _dense_pre_body,
    out_shape=jax.ShapeDtypeStruct((N, D), jnp.float32),
)


def _dense_post_body(aggp_ref, dinp_ref, b_ref, a1_ref, g_ref, be_ref, a2_ref,
                     o_ref):
    agg = aggp_ref[0] + aggp_ref[1]
    deg = dinp_ref[0, :] + dinp_ref[1, :]
    nd = lax.rsqrt(jnp.maximum(deg, 1.0))
    out = agg * nd[:, None] + b_ref[...]
    a1 = a1_ref[0]
    out = jnp.where(out >= 0, out, a1 * out)
    mean = jnp.mean(out, axis=0)
    cent = out - mean
    var = jnp.mean(cent * cent, axis=0)
    hn = cent * lax.rsqrt(var + 1e-5) * g_ref[...] + be_ref[...]
    a2 = a2_ref[0]
    o_ref[...] = jnp.where(hn >= 0, hn, a2 * hn)


_dense_post = pl.pallas_call(
    _dense_post_body,
    out_shape=jax.ShapeDtypeStruct((N, D), jnp.float32),
)


def kernel(heat, edge_index, W, b, a1, gamma, beta, a2):
    ei = edge_index.astype(jnp.int32)
    src_r = ei[0].reshape(NCHUNK, CH)
    dst_r = ei[1].reshape(NCHUNK, CH)
    dout_p, din_p = _hist(src_r, dst_r)
    z = _dense_pre(heat, W, dout_p.reshape(NC, N))
    agg_p = _agg(z, src_r, dst_r)
    return _dense_post(agg_p, din_p.reshape(NC, N), b, a1, gamma, beta, a2)


# speedup vs baseline: 1.1786x; 1.0580x over previous
import functools

import jax
import jax.numpy as jnp
from jax import lax
from jax.experimental import pallas as pl
from jax.experimental.pallas import tpu as pltpu
from jax.experimental.pallas import tpu_sc as plsc

N = 10000
E = 320000
D = 128
NC, NS = 2, 16
NW = NC * NS
CH = 128
NCHUNK = E // CH
NBUF = 3
KMAX = 81

ROWS_A, ROWS_B = 632, 520

_mesh = plsc.VectorSubcoreMesh(core_axis_name="c", subcore_axis_name="s")


def _tile_rows(s):
    return (s * ROWS_A, ROWS_A if s < NS - 1 else ROWS_B)


@functools.partial(
    pl.kernel,
    out_type=(jax.ShapeDtypeStruct((NC * N,), jnp.float32),
              jax.ShapeDtypeStruct((NC * N,), jnp.float32)),
    mesh=_mesh,
    scratch_types=[
        pltpu.VMEM((NBUF, CH), jnp.int32),
        pltpu.VMEM((NBUF, CH), jnp.int32),
        pltpu.VMEM((CH,), jnp.float32),
        pltpu.VMEM((ROWS_A + 8,), jnp.float32),
        pltpu.VMEM_SHARED((N,), jnp.float32),
        pltpu.VMEM_SHARED((N,), jnp.float32),
        pltpu.SemaphoreType.DMA((NBUF,)),
        pltpu.SemaphoreType.DMA((NBUF,)),
        pltpu.SemaphoreType.DMA((NBUF,)),
        pltpu.SemaphoreType.DMA((NBUF,)),
    ],
)
def _hist(ei_h, dout_h, din_h, src_v, dst_v, ones_v, stage_v,
          dout_sh, din_sh, sem_i, sem_j, sem_h, sem_n):
    c = lax.axis_index("c")
    s = lax.axis_index("s")
    w = s * NC + c

    one16 = jnp.ones((16,), jnp.float32)
    zero16 = jnp.zeros((16,), jnp.float32)
    for i in range(CH // 16):
        ones_v[pl.ds(i * 16, 16)] = one16
    for i in range((ROWS_A + 8) // 16):
        stage_v[pl.ds(i * 16, 16)] = zero16

    def valid(k):
        return (k >= 0) & (k < KMAX) & (k * NW + w < NCHUNK)

    def issue_idx(k, p):
        @pl.when(valid(k))
        def _():
            pltpu.async_copy(ei_h.at[pl.ds((k * NW + w) * CH, CH)],
                             src_v.at[p], sem_i.at[p])
            pltpu.async_copy(ei_h.at[pl.ds(E + (k * NW + w) * CH, CH)],
                             dst_v.at[p], sem_j.at[p])

    issue_idx(0, 0)

    for sv in range(NS):
        lo, sz = _tile_rows(sv)

        @pl.when(s == sv)
        def _():
            pltpu.sync_copy(stage_v.at[pl.ds(0, sz)], dout_sh.at[pl.ds(lo, sz)])
            pltpu.sync_copy(stage_v.at[pl.ds(0, sz)], din_sh.at[pl.ds(lo, sz)])

    plsc.subcore_barrier()

    def wait_idx(k, p):
        @pl.when(valid(k))
        def _():
            pltpu.make_async_copy(ei_h.at[pl.ds((k * NW + w) * CH, CH)],
                                  src_v.at[p], sem_i.at[p]).wait()
            pltpu.make_async_copy(ei_h.at[pl.ds(E + (k * NW + w) * CH, CH)],
                                  dst_v.at[p], sem_j.at[p]).wait()

    def issue_hist(k, p):
        @pl.when(valid(k))
        def _():
            pltpu.async_copy(ones_v, dout_sh.at[src_v.at[p]], sem_h.at[p],
                             add=True)
            pltpu.async_copy(ones_v, din_sh.at[dst_v.at[p]], sem_n.at[p],
                             add=True)

    def wait_hist(k, p):
        @pl.when(valid(k))
        def _():
            pltpu.make_async_copy(ones_v, dout_sh.at[src_v.at[p]],
                                  sem_h.at[p]).wait()
            pltpu.make_async_copy(ones_v, din_sh.at[dst_v.at[p]],
                                  sem_n.at[p]).wait()

    def body(i, carry):
        for sub in range(NBUF):
            k = i * NBUF + sub
            p = sub
            wait_hist(k - 2, (p + 1) % NBUF)
            issue_idx(k + 1, (p + 1) % NBUF)
            wait_idx(k, p)
            issue_hist(k, p)
        return carry

    lax.fori_loop(0, KMAX // NBUF, body, 0)
    for kk in range(KMAX - 2, KMAX):
        wait_hist(kk, kk % NBUF)

    plsc.subcore_barrier()

    for sv in range(NS):
        lo, sz = _tile_rows(sv)

        @pl.when(s == sv)
        def _():
            pltpu.sync_copy(dout_sh.at[pl.ds(lo, sz)], stage_v.at[pl.ds(0, sz)])
            pltpu.sync_copy(stage_v.at[pl.ds(0, sz)],
                            dout_h.at[pl.ds(c * N + lo, sz)])
            pltpu.sync_copy(din_sh.at[pl.ds(lo, sz)], stage_v.at[pl.ds(0, sz)])
            pltpu.sync_copy(stage_v.at[pl.ds(0, sz)],
                            din_h.at[pl.ds(c * N + lo, sz)])


@functools.partial(
    pl.kernel,
    out_type=jax.ShapeDtypeStruct((NC, N, D), jnp.float32),
    mesh=_mesh,
    scratch_types=[
        pltpu.VMEM((NBUF, CH), jnp.int32),
        pltpu.VMEM((NBUF, CH), jnp.int32),
        pltpu.VMEM((NBUF, CH, D), jnp.float32),
        pltpu.VMEM_SHARED((N, D), jnp.float32),
        pltpu.SemaphoreType.DMA((NBUF,)),
        pltpu.SemaphoreType.DMA((NBUF,)),
        pltpu.SemaphoreType.DMA((NBUF,)),
        pltpu.SemaphoreType.DMA((NBUF,)),
    ],
)
def _agg(zp_h, ei_h, out_h, src_v, dst_v, rows_v, agg_sh,
         sem_s, sem_d, sem_g, sem_a):
    c = lax.axis_index("c")
    s = lax.axis_index("s")
    w = s * NC + c

    zero16 = jnp.zeros((16,), jnp.float32)
    for r in range(8):
        for q in range(D // 16):
            rows_v[0, r, pl.ds(q * 16, 16)] = zero16

    def valid(k):
        return (k >= 0) & (k < KMAX) & (k * NW + w < NCHUNK)

    def issue_idx(k, p):
        @pl.when(valid(k))
        def _():
            pltpu.async_copy(ei_h.at[pl.ds((k * NW + w) * CH, CH)],
                             src_v.at[p], sem_s.at[p])
            pltpu.async_copy(ei_h.at[pl.ds(E + (k * NW + w) * CH, CH)],
                             dst_v.at[p], sem_d.at[p])

    issue_idx(0, 0)

    for sv in range(NS):
        lo, sz = _tile_rows(sv)

        @pl.when(s == sv)
        def _():
            def zbody(i, carry):
                pltpu.sync_copy(rows_v.at[0, pl.ds(0, 8)],
                                agg_sh.at[pl.ds(lo + i * 8, 8)])
                return carry

            lax.fori_loop(0, sz // 8, zbody, 0)

    plsc.subcore_barrier()

    def wait_idx(k, p):
        @pl.when(valid(k))
        def _():
            pltpu.make_async_copy(ei_h.at[pl.ds((k * NW + w) * CH, CH)],
                                  src_v.at[p], sem_s.at[p]).wait()
            pltpu.make_async_copy(ei_h.at[pl.ds(E + (k * NW + w) * CH, CH)],
                                  dst_v.at[p], sem_d.at[p]).wait()

    def issue_gather(k, p):
        @pl.when(valid(k))
        def _():
            pltpu.async_copy(zp_h.at[src_v.at[p]], rows_v.at[p], sem_g.at[p])

    def wait_gather(k, p):
        @pl.when(valid(k))
        def _():
            pltpu.make_async_copy(zp_h.at[src_v.at[p]], rows_v.at[p],
                                  sem_g.at[p]).wait()

    def issue_scat(k, p):
        @pl.when(valid(k))
        def _():
            pltpu.async_copy(rows_v.at[p], agg_sh.at[dst_v.at[p]], sem_a.at[p],
                             add=True)

    def wait_scat(k, p):
        @pl.when(valid(k))
        def _():
            pltpu.make_async_copy(rows_v.at[p], agg_sh.at[dst_v.at[p]],
                                  sem_a.at[p]).wait()

    def body(i, carry):
        for sub in range(NBUF):
            k = i * NBUF + sub
            p = sub
            pn = (p + 1) % NBUF
            pb = (p + 2) % NBUF
            wait_scat(k - 2, pn)
            issue_idx(k + 1, pn)
            wait_idx(k, p)
            issue_gather(k, p)
            wait_gather(k - 1, pb)
            issue_scat(k - 1, pb)
        return carry

    lax.fori_loop(0, KMAX // NBUF, body, 0)
    kk = KMAX - 1
    wait_gather(kk, kk % NBUF)
    issue_scat(kk, kk % NBUF)
    for kk in range(KMAX - 2, KMAX):
        wait_scat(kk, kk % NBUF)

    plsc.subcore_barrier()

    for sv in range(NS):
        lo, sz = _tile_rows(sv)

        @pl.when(s == sv)
        def _():
            pltpu.sync_copy(agg_sh.at[pl.ds(lo, sz)],
                            out_h.at[c, pl.ds(lo, sz)])


def _dense_pre_body(x_ref, w_ref, dparts_ref, o_ref):
    deg = dparts_ref[0, :] + dparts_ref[1, :]
    ns = lax.rsqrt(jnp.maximum(deg, 1.0))
    z = jnp.dot(x_ref[...], w_ref[...], preferred_element_type=jnp.float32)
    o_ref[...] = z * ns[:, None]


_dense_pre = pl.pallas_call(
    _dense_pre_body,
    out_shape=jax.ShapeDtypeStruct((N, D), jnp.float32),
)


def _dense_post_body(aggp_ref, dinp_ref, b_ref, a1_ref, g_ref, be_ref, a2_ref,
                     o_ref):
    agg = aggp_ref[0] + aggp_ref[1]
    deg = dinp_ref[0, :] + dinp_ref[1, :]
    nd = lax.rsqrt(jnp.maximum(deg, 1.0))
    out = agg * nd[:, None] + b_ref[...]
    a1 = a1_ref[0]
    out = jnp.where(out >= 0, out, a1 * out)
    mean = jnp.mean(out, axis=0)
    cent = out - mean
    var = jnp.mean(cent * cent, axis=0)
    hn = cent * lax.rsqrt(var + 1e-5) * g_ref[...] + be_ref[...]
    a2 = a2_ref[0]
    o_ref[...] = jnp.where(hn >= 0, hn, a2 * hn)


_dense_post = pl.pallas_call(
    _dense_post_body,
    out_shape=jax.ShapeDtypeStruct((N, D), jnp.float32),
)


def kernel(heat, edge_index, W, b, a1, gamma, beta, a2):
    ei_flat = edge_index.astype(jnp.int32).reshape(2 * E)
    dout_p, din_p = _hist(ei_flat)
    z = _dense_pre(heat, W, dout_p.reshape(NC, N))
    agg_p = _agg(z, ei_flat)
    return _dense_post(agg_p, din_p.reshape(NC, N), b, a1, gamma, beta, a2)
